# baseline (speedup 1.0000x reference)
import functools
import jax
import jax.numpy as jnp
from jax import lax
from jax.experimental import pallas as pl
from jax.experimental.pallas import tpu as pltpu
from jax.experimental.pallas import tpu_sc as plsc

_K = 16
_BR = 256


def _score_block(xq_ref, xk_ref, sqq_ref, skk_ref, out_ref):
    qn = xq_ref[0]
    kn = xk_ref[0]
    sqq = sqq_ref[0]
    skk = skk_ref[0]
    s = lax.dot_general(
        qn, kn, (((1,), (1,)), ((), ())), preferred_element_type=jnp.float32
    )
    d2 = sqq + skk - 2.0 * s
    out_ref[0] = -jnp.maximum(d2, 0.0)


def _scores(xt, sq):
    b, n, d = xt.shape
    return pl.pallas_call(
        _score_block,
        grid=(b, n // _BR),
        in_specs=[
            pl.BlockSpec((1, _BR, d), lambda bi, i: (bi, i, 0)),
            pl.BlockSpec((1, n, d), lambda bi, i: (bi, 0, 0)),
            pl.BlockSpec((1, _BR, 1), lambda bi, i: (bi, i, 0)),
            pl.BlockSpec((1, 1, n), lambda bi, i: (bi, 0, 0)),
        ],
        out_specs=pl.BlockSpec((1, _BR, n), lambda bi, i: (bi, i, 0)),
        out_shape=jax.ShapeDtypeStruct((b, n, n), jnp.float32),
    )(xt, xt, sq[:, :, None], sq[:, None, :])


def _sc_topk(scores2d):
    rows, n = scores2d.shape
    info = plsc.get_sparse_core_info()
    nw = info.num_cores * info.num_subcores
    rows_per_w = rows // nw
    tile_rows = 4
    mesh = plsc.VectorSubcoreMesh(core_axis_name="c", subcore_axis_name="s")

    @functools.partial(
        pl.kernel,
        mesh=mesh,
        compiler_params=pltpu.CompilerParams(needs_layout_passes=False),
        out_type=jax.ShapeDtypeStruct((rows * _K,), jnp.int32),
        scratch_types=[
            pltpu.VMEM((tile_rows * n,), jnp.float32),
            pltpu.VMEM((rows_per_w * _K,), jnp.int32),
        ],
    )
    def k(scores_hbm, out_hbm, rows_v, out_v):
        wid = lax.axis_index("s") * info.num_cores + lax.axis_index("c")
        base = wid * rows_per_w

        def tile_body(tb, acc):
            r0 = (base + tb * tile_rows) * n
            pltpu.sync_copy(scores_hbm.at[pl.ds(r0, tile_rows * n)], rows_v)

            def row_body(r, acc2):
                lane16 = lax.iota(jnp.int32, 16)

                def chunk_body(c, carry):
                    lv, li, lmin = carry
                    cv = rows_v[pl.ds(r * n + c * 16, 16)]
                    cmax = lax.reduce_max(cv, axes=(0,))

                    def merge(op):
                        lv_, li_ = op
                        ci = lane16 + c * 16
                        csort, cisort = plsc.sort_key_val(cv, ci)
                        sel = lv_ >= csort
                        mv = jnp.where(sel, lv_, csort)
                        mi = jnp.where(sel, li_, cisort)
                        nv, ni = plsc.sort_key_val(mv, mi, descending=True)
                        return nv, ni, lax.reduce_min(nv, axes=(0,))

                    def keep(op):
                        return op[0], op[1], lmin

                    return lax.cond(cmax > lmin, merge, keep, (lv, li))

                v0 = rows_v[pl.ds(r * n, 16)]
                lv0, li0 = plsc.sort_key_val(v0, lane16, descending=True)
                lmin0 = lax.reduce_min(lv0, axes=(0,))
                lv, li, _ = lax.fori_loop(
                    1, n // 16, chunk_body, (lv0, li0, lmin0)
                )
                out_v[pl.ds((tb * tile_rows + r) * _K, _K)] = li
                return acc2

            return lax.fori_loop(0, tile_rows, row_body, acc)

        lax.fori_loop(0, rows_per_w // tile_rows, tile_body, 0)
        pltpu.sync_copy(out_v, out_hbm.at[pl.ds(base * _K, rows_per_w * _K)])

    out = k(scores2d.reshape(-1))
    return out.reshape(rows, _K)


def kernel(x, relative_pos):
    del relative_pos
    norm = jnp.sqrt(jnp.sum(x * x, axis=1, keepdims=True))
    xn = x / jnp.maximum(norm, 1e-12)
    xt = jnp.squeeze(jnp.transpose(xn, (0, 2, 1, 3)), -1)
    b, n, d = xt.shape
    sq = jnp.sum(xt * xt, axis=-1)
    sc = _scores(xt, sq)
    nn_flat = _sc_topk(sc.reshape(b * n, n))
    nn_idx = nn_flat.reshape(b, n, _K)
    center_idx = jnp.broadcast_to(
        jnp.arange(n, dtype=jnp.int32)[None, :, None], (b, n, _K)
    )
    return jnp.stack((nn_idx, center_idx), axis=0)


# speedup vs baseline: 6.7006x; 6.7006x over previous
import jax
import jax.numpy as jnp
from jax.experimental import pallas as pl

_K = 16
_BR = 256
_L = 128


def _oem_pairs(nn):
    pairs = []

    def merge(lo, m, r):
        step = r * 2
        if step < m:
            merge(lo, m, step)
            merge(lo + r, m, step)
            for i in range(lo + r, lo + m - r, step):
                pairs.append((i, i + r))
        else:
            pairs.append((lo, lo + r))

    def sort(lo, m):
        if m > 1:
            h = m // 2
            sort(lo, h)
            sort(lo + h, h)
            merge(lo, m, 1)

    sort(0, nn)
    return pairs


def _pruned_pairs(nn, need_hi):
    needed = set(range(need_hi + 1))
    kept = []
    for i, j in reversed(_oem_pairs(nn)):
        if i in needed or j in needed:
            kept.append((i, j))
            needed.add(i)
            needed.add(j)
    return list(reversed(kept))


def _select(scores, n):
    br = scores.shape[0]
    depths = n // _L
    lane = jax.lax.broadcasted_iota(jnp.int32, (br, _L), 1)
    v = [scores[:, d * _L : (d + 1) * _L] for d in range(depths)]
    dep = [jnp.full((br, _L), d, jnp.int32) for d in range(depths)]

    for i, j in _pruned_pairs(depths, _K - 1):
        sw = v[j] < v[i]
        v[i], v[j] = jnp.where(sw, v[j], v[i]), jnp.where(sw, v[i], v[j])
        dep[i], dep[j] = (
            jnp.where(sw, dep[j], dep[i]),
            jnp.where(sw, dep[i], dep[j]),
        )

    big = jnp.int32(n)
    h, hd = v[0], dep[0]
    hp = jnp.zeros((br, _L), jnp.int32)
    outs = []
    for t in range(_K):
        m = jnp.min(h, axis=1, keepdims=True)
        cand = jnp.where(h == m, hd * _L + lane, big)
        c = jnp.min(cand, axis=1, keepdims=True)
        outs.append(c)
        if t < _K - 1:
            win = cand == c
            hp = hp + win.astype(jnp.int32)
            for p in range(1, min(t + 2, depths)):
                at = win & (hp == p)
                h = jnp.where(at, v[p], h)
                hd = jnp.where(at, dep[p], hd)
    return jnp.concatenate(outs, axis=1)


_G = 64


def _knn_block(xq_ref, xk_ref, sqq_ref, skk_ref, out_ref):
    qn = xq_ref[0]
    kn = xk_ref[0]
    n = kn.shape[0]
    br = qn.shape[0]

    sqq = sqq_ref[0]
    skk = skk_ref[0]

    s2 = jax.lax.dot_general(
        qn, kn, (((1,), (1,)), ((), ())), preferred_element_type=jnp.float32
    )
    d2 = sqq + skk - s2
    scores = jnp.maximum(d2, 0.0)

    outs = [
        _select(scores[g * _G : (g + 1) * _G], n) for g in range(br // _G)
    ]
    out_ref[0] = jnp.concatenate(outs, axis=0)


def kernel(x, relative_pos):
    del relative_pos
    norm = jnp.sqrt(jnp.sum(x * x, axis=1, keepdims=True))
    xn = x / jnp.maximum(norm, 1e-12)
    xt = jnp.squeeze(jnp.transpose(xn, (0, 2, 1, 3)), -1)
    b, n, d = xt.shape
    sq = jnp.sum(xt * xt, axis=-1)
    sq_q = sq[:, :, None]
    sq_k = sq[:, None, :]

    nn_idx = pl.pallas_call(
        _knn_block,
        grid=(b, n // _BR),
        in_specs=[
            pl.BlockSpec((1, _BR, d), lambda bi, i: (bi, i, 0)),
            pl.BlockSpec((1, n, d), lambda bi, i: (bi, 0, 0)),
            pl.BlockSpec((1, _BR, 1), lambda bi, i: (bi, i, 0)),
            pl.BlockSpec((1, 1, n), lambda bi, i: (bi, 0, 0)),
        ],
        out_specs=pl.BlockSpec((1, _BR, _K), lambda bi, i: (bi, i, 0)),
        out_shape=jax.ShapeDtypeStruct((b, n, _K), jnp.int32),
    )(2.0 * xt, xt, sq_q, sq_k)

    center_idx = jnp.broadcast_to(
        jnp.arange(n, dtype=jnp.int32)[None, :, None], (b, n, _K)
    )
    return jnp.stack((nn_idx, center_idx), axis=0)


# speedup vs baseline: 7.9630x; 1.1884x over previous
import jax
import jax.numpy as jnp
from jax.experimental import pallas as pl

_K = 16
_BR = 512
_L = 128


def _oem_pairs(nn):
    pairs = []

    def merge(lo, m, r):
        step = r * 2
        if step < m:
            merge(lo, m, step)
            merge(lo + r, m, step)
            for i in range(lo + r, lo + m - r, step):
                pairs.append((i, i + r))
        else:
            pairs.append((lo, lo + r))

    def sort(lo, m):
        if m > 1:
            h = m // 2
            sort(lo, h)
            sort(lo + h, h)
            merge(lo, m, 1)

    sort(0, nn)
    return pairs


def _pruned_pairs(nn, need_hi):
    needed = set(range(need_hi + 1))
    kept = []
    for i, j in reversed(_oem_pairs(nn)):
        if i in needed or j in needed:
            kept.append((i, j))
            needed.add(i)
            needed.add(j)
    return list(reversed(kept))


def _select(scores, n):
    br = scores.shape[0]
    depths = n // _L
    lane = jax.lax.broadcasted_iota(jnp.int32, (br, _L), 1)
    v = [scores[:, d * _L : (d + 1) * _L] for d in range(depths)]
    dep = [jnp.full((br, _L), d, jnp.int32) for d in range(depths)]

    for i, j in _pruned_pairs(depths, _K - 1):
        sw = v[j] < v[i]
        v[i], v[j] = jnp.where(sw, v[j], v[i]), jnp.where(sw, v[i], v[j])
        dep[i], dep[j] = (
            jnp.where(sw, dep[j], dep[i]),
            jnp.where(sw, dep[i], dep[j]),
        )

    big = jnp.int32(n)
    h, hd = v[0], dep[0]
    hp = jnp.zeros((br, _L), jnp.int32)
    outs = []
    for t in range(_K):
        m = jnp.min(h, axis=1, keepdims=True)
        cand = jnp.where(h == m, hd * _L + lane, big)
        c = jnp.min(cand, axis=1, keepdims=True)
        outs.append(c)
        if t < _K - 1:
            win = cand == c
            hp = hp + win.astype(jnp.int32)
            for p in range(1, min(t + 2, depths)):
                at = win & (hp == p)
                h = jnp.where(at, v[p], h)
                hd = jnp.where(at, dep[p], hd)
    return jnp.concatenate(outs, axis=1)


_G = 64


def _knn_block(xq_ref, xk_ref, sqq_ref, skk_ref, out_ref):
    qn = xq_ref[0]
    kn = xk_ref[0]
    n = kn.shape[0]
    br = qn.shape[0]

    sqq = sqq_ref[0]
    skk = skk_ref[0]

    s2 = jax.lax.dot_general(
        qn, kn, (((1,), (1,)), ((), ())), preferred_element_type=jnp.float32
    )
    d2 = sqq + skk - s2
    scores = jnp.maximum(d2, 0.0)

    outs = [
        _select(scores[g * _G : (g + 1) * _G], n) for g in range(br // _G)
    ]
    out_ref[0] = jnp.concatenate(outs, axis=0)


def kernel(x, relative_pos):
    del relative_pos
    norm = jnp.sqrt(jnp.sum(x * x, axis=1, keepdims=True))
    xn = x / jnp.maximum(norm, 1e-12)
    xt = jnp.squeeze(jnp.transpose(xn, (0, 2, 1, 3)), -1)
    b, n, d = xt.shape
    sq = jnp.sum(xt * xt, axis=-1)
    sq_q = sq[:, :, None]
    sq_k = sq[:, None, :]

    nn_idx = pl.pallas_call(
        _knn_block,
        grid=(b, n // _BR),
        in_specs=[
            pl.BlockSpec((1, _BR, d), lambda bi, i: (bi, i, 0)),
            pl.BlockSpec((1, n, d), lambda bi, i: (bi, 0, 0)),
            pl.BlockSpec((1, _BR, 1), lambda bi, i: (bi, i, 0)),
            pl.BlockSpec((1, 1, n), lambda bi, i: (bi, 0, 0)),
        ],
        out_specs=pl.BlockSpec((1, _BR, _K), lambda bi, i: (bi, i, 0)),
        out_shape=jax.ShapeDtypeStruct((b, n, _K), jnp.int32),
    )(2.0 * xt, xt, sq_q, sq_k)

    center_idx = jnp.broadcast_to(
        jnp.arange(n, dtype=jnp.int32)[None, :, None], (b, n, _K)
    )
    return jnp.stack((nn_idx, center_idx), axis=0)
